# baseline (speedup 1.0000x reference)
import jax
import jax.numpy as jnp
from jax.experimental import pallas as pl

N = 1000000
BLOCK = 8000


def _body(n_ref, p_ref, f_ref, o_ref):
    n = n_ref[...]
    p = p_ref[...]
    f = f_ref[...]
    lp = (p - jnp.floor(p)) - 0.5
    dis = jnp.sum(lp * n, axis=1, keepdims=True)
    o_ref[...] = jnp.concatenate([n, dis, f], axis=1)


def kernel(normals, points, features):
    grid = (N // BLOCK,)
    return pl.pallas_call(
        _body,
        grid=grid,
        in_specs=[
            pl.BlockSpec((BLOCK, 3), lambda i: (i, 0)),
            pl.BlockSpec((BLOCK, 3), lambda i: (i, 0)),
            pl.BlockSpec((BLOCK, 4), lambda i: (i, 0)),
        ],
        out_specs=pl.BlockSpec((BLOCK, 8), lambda i: (i, 0)),
        out_shape=jax.ShapeDtypeStruct((N, 8), jnp.float32),
    )(normals, points, features)
